# Initial kernel scaffold; baseline (speedup 1.0000x reference)
import jax
import jax.numpy as jnp
from jax.experimental import pallas as pl


def kernel(x, edge_index, W0l, W0r, b0, W1l, W1r, b1, Wel, Wer, be, Wpl, Wpr, bp, W3l, W3r, b3, W4l, W4r, b4, Wfl, Wfr, bf, Wql, Wqr, bq, Wro, bro):
    raise NotImplementedError("write your pallas kernel here")


# speedup vs baseline: 8.8218x; 8.8218x over previous
import functools

import jax
import jax.numpy as jnp
from jax import lax
from jax.experimental import pallas as pl
from jax.experimental.pallas import tpu as pltpu
from jax.experimental.pallas import tpu_sc as plsc

N = 10000
E = 160000
NC = 2
NS = 16
NW = NC * NS
GB = 40
ROWS_PER_W = E // GB // NW
SLAB = 640
NPAD = NS * SLAB

_f32 = jnp.float32


def _make_seg_kernel(width):
    mesh = plsc.VectorSubcoreMesh(core_axis_name="c", subcore_axis_name="s")

    @functools.partial(
        pl.kernel,
        mesh=mesh,
        compiler_params=pltpu.CompilerParams(use_tc_tiling_on_sc=False),
        out_type=jax.ShapeDtypeStruct((NC, NPAD, width), _f32),
        scratch_types=[
            pltpu.VMEM((ROWS_PER_W, GB), jnp.int32),
            pltpu.VMEM((ROWS_PER_W, GB), jnp.int32),
            pltpu.VMEM((GB, width), _f32),
            pltpu.VMEM((SLAB, width), _f32),
            pltpu.VMEM_SHARED((NPAD, width), _f32),
            pltpu.SemaphoreType.DMA,
        ],
    )
    def seg(table, src3d, dst3d, zeros, out, srcv, dstv, rowsv, stage, accum, sem):
        cid = lax.axis_index("c")
        sid = lax.axis_index("s")
        wid = sid * NC + cid

        pltpu.sync_copy(zeros, stage)
        pltpu.sync_copy(stage, accum.at[pl.ds(sid * SLAB, SLAB)])
        plsc.subcore_barrier()

        pltpu.sync_copy(src3d.at[wid], srcv)
        pltpu.sync_copy(dst3d.at[wid], dstv)

        def chunk(j, carry):
            pltpu.async_copy(table.at[srcv.at[j]], rowsv, sem).wait()
            pltpu.sync_copy(rowsv, accum.at[dstv.at[j]], add=True)
            return carry

        lax.fori_loop(0, ROWS_PER_W, chunk, 0)
        plsc.subcore_barrier()

        pltpu.sync_copy(accum.at[pl.ds(sid * SLAB, SLAB)], stage)
        pltpu.sync_copy(stage, out.at[cid, pl.ds(sid * SLAB, SLAB)])

    return seg


_seg32 = _make_seg_kernel(32)
_seg16 = _make_seg_kernel(16)


def _k1_body(x_ref, w0l_ref, w0r_ref, b0_ref, xaug_ref, xr_ref):
    x = x_ref[...]
    xl = jnp.dot(x, w0l_ref[...], preferred_element_type=_f32)
    ones = jnp.ones((x.shape[0], 1), _f32)
    zeros = jnp.zeros((x.shape[0], 15), _f32)
    xaug_ref[...] = jnp.concatenate([xl, ones, zeros], axis=1)
    xr_ref[...] = jnp.dot(x, w0r_ref[...], preferred_element_type=_f32) + b0_ref[...]


def _k3_body(s0_ref, xr_ref, w1l_ref, w1r_ref, b1_ref, h0l_ref, h0r_ref, invd_ref):
    s0 = s0_ref[0] + s0_ref[1]
    agg0 = s0[:N, :16]
    deg = s0[:N, 16:17]
    invd = 1.0 / jnp.maximum(deg, 1.0)
    h0 = jnp.maximum(agg0 * invd + xr_ref[...], 0.0)
    h0l_ref[...] = jnp.dot(h0, w1l_ref[...], preferred_element_type=_f32)
    h0r_ref[...] = jnp.dot(h0, w1r_ref[...], preferred_element_type=_f32) + b1_ref[...]
    invd_ref[...] = invd


def _k5_body(a1_ref, invd_ref, h0r_ref, h1_ref):
    agg1 = a1_ref[0][:N] + a1_ref[1][:N]
    h1_ref[...] = jnp.maximum(agg1 * invd_ref[...] + h0r_ref[...], 0.0)


def _k7_body(a2_ref, invd_ref, h1_ref, wel_ref, wer_ref, be_ref, wpl_ref, wpr_ref,
             bp_ref, w3l_ref, w3r_ref, b3_ref, w4l_ref, w4r_ref, b4_ref,
             wfl_ref, wfr_ref, bf_ref, wro_ref, bro_ref, out_ref):
    h1 = h1_ref[...]
    mean2 = (a2_ref[0][:N] + a2_ref[1][:N]) * invd_ref[...]
    z = jnp.maximum(jnp.dot(mean2, wel_ref[...], preferred_element_type=_f32)
                    + jnp.dot(h1, wer_ref[...], preferred_element_type=_f32)
                    + be_ref[...], 0.0)
    sl = (jnp.dot(mean2, wpl_ref[...], preferred_element_type=_f32)
          + jnp.dot(h1, wpr_ref[...], preferred_element_type=_f32) + bp_ref[...])
    m = jnp.max(sl, axis=1, keepdims=True)
    e = jnp.exp(sl - m)
    s = e / jnp.sum(e, axis=1, keepdims=True)

    hp = lax.dot_general(s, z, (((0,), (0,)), ((), ())),
                         preferred_element_type=_f32)

    mh = jnp.mean(hp, axis=0, keepdims=True)
    h3 = jnp.maximum(jnp.dot(mh, w3l_ref[...], preferred_element_type=_f32)
                     + jnp.dot(hp, w3r_ref[...], preferred_element_type=_f32)
                     + b3_ref[...], 0.0)
    mh3 = jnp.mean(h3, axis=0, keepdims=True)
    h4 = jnp.maximum(jnp.dot(mh3, w4l_ref[...], preferred_element_type=_f32)
                     + jnp.dot(h3, w4r_ref[...], preferred_element_type=_f32)
                     + b4_ref[...], 0.0)
    mh4 = jnp.mean(h4, axis=0, keepdims=True)
    fz = jnp.maximum(jnp.dot(mh4, wfl_ref[...], preferred_element_type=_f32)
                     + jnp.dot(h4, wfr_ref[...], preferred_element_type=_f32)
                     + bf_ref[...], 0.0)
    xfin = jnp.sum(fz, axis=0, keepdims=True)
    out_ref[...] = (jnp.dot(xfin, wro_ref[...], preferred_element_type=_f32)
                    + bro_ref[...])


def _tc_call(body, out_shapes):
    return pl.pallas_call(body, out_shape=out_shapes)


def kernel(x, edge_index, W0l, W0r, b0, W1l, W1r, b1, Wel, Wer, be, Wpl, Wpr,
           bp, W3l, W3r, b3, W4l, W4r, b4, Wfl, Wfr, bf, Wql, Wqr, bq, Wro, bro):
    src3d = edge_index[0].reshape(NW, ROWS_PER_W, GB)
    dst3d = edge_index[1].reshape(NW, ROWS_PER_W, GB)
    z32 = jnp.zeros((SLAB, 32), _f32)
    z16 = jnp.zeros((SLAB, 16), _f32)

    xaug, xr = _tc_call(_k1_body, [
        jax.ShapeDtypeStruct((N, 32), _f32),
        jax.ShapeDtypeStruct((N, 16), _f32),
    ])(x, W0l, W0r, b0.reshape(1, 16))

    s0 = _seg32(xaug, src3d, dst3d, z32)

    h0l, h0r, invd = _tc_call(_k3_body, [
        jax.ShapeDtypeStruct((N, 16), _f32),
        jax.ShapeDtypeStruct((N, 16), _f32),
        jax.ShapeDtypeStruct((N, 1), _f32),
    ])(s0, xr, W1l, W1r, b1.reshape(1, 16))

    a1 = _seg16(h0l, src3d, dst3d, z16)

    (h1,) = _tc_call(_k5_body, [jax.ShapeDtypeStruct((N, 16), _f32)])(a1, invd, h0r)

    a2 = _seg16(h1, src3d, dst3d, z16)

    (out,) = _tc_call(_k7_body, [jax.ShapeDtypeStruct((1, 6), _f32)])(
        a2, invd, h1,
        Wel, Wer, be.reshape(1, 10), Wpl, Wpr, bp.reshape(1, 10),
        W3l, W3r, b3.reshape(1, 16), W4l, W4r, b4.reshape(1, 16),
        Wfl, Wfr, bf.reshape(1, 16), Wro, bro.reshape(1, 6))
    return out
